# baseline (speedup 1.0000x reference)
import functools

import jax
import jax.numpy as jnp
from jax import lax
from jax.experimental import pallas as pl
from jax.experimental.pallas import tpu as pltpu
from jax.experimental.pallas import tpu_sc as plsc

D = 128
N_TOK = 8192
NC = 2
NS = 16
NW = NC * NS
PER_W = N_TOK // NW
CHUNK = 128
LANES = 16

_mesh = plsc.VectorSubcoreMesh(core_axis_name="c", subcore_axis_name="s")


def _add_rows(a, b):
    @plsc.parallel_loop(0, CHUNK, step=1, unroll=4)
    def body(i):
        for j in range(D // LANES):
            sl = pl.ds(j * LANES, LANES)
            plsc.addupdate(a.at[i, sl], b[i, sl])


@functools.partial(
    pl.kernel,
    out_type=jax.ShapeDtypeStruct((N_TOK, D), jnp.float32),
    mesh=_mesh,
    scratch_types=[
        pltpu.VMEM((2, CHUNK), jnp.int32),
        pltpu.VMEM((2, CHUNK), jnp.int32),
        pltpu.VMEM((CHUNK, D), jnp.float32),
        pltpu.VMEM((CHUNK, D), jnp.float32),
        pltpu.VMEM((CHUNK, D), jnp.float32),
        pltpu.VMEM((CHUNK, D), jnp.float32),
        pltpu.SemaphoreType.DMA,
        pltpu.SemaphoreType.DMA,
        pltpu.SemaphoreType.DMA,
        pltpu.SemaphoreType.DMA,
        pltpu.SemaphoreType.DMA,
        pltpu.SemaphoreType.DMA,
        pltpu.SemaphoreType.DMA,
    ],
)
def _embed(idx_hbm, wte_hbm, wpe_hbm, out_hbm,
           ti_v, pi_v, a0, b0, a1, b1,
           si0, si1, sa0, sb0, sa1, sb1, so):
    wid = lax.axis_index("s") * NC + lax.axis_index("c")
    r = wid * 2
    ci0 = pltpu.async_copy(idx_hbm.at[pl.ds(r, 2)], ti_v, si0)
    ci1 = pltpu.async_copy(idx_hbm.at[pl.ds(NW * 2 + r, 2)], pi_v, si1)
    ci0.wait()
    ci1.wait()
    ga0 = pltpu.async_copy(wte_hbm.at[ti_v.at[0]], a0, sa0)
    gb0 = pltpu.async_copy(wpe_hbm.at[pi_v.at[0]], b0, sb0)
    ga1 = pltpu.async_copy(wte_hbm.at[ti_v.at[1]], a1, sa1)
    gb1 = pltpu.async_copy(wpe_hbm.at[pi_v.at[1]], b1, sb1)
    base = wid * PER_W
    ga0.wait()
    gb0.wait()
    _add_rows(a0, b0)
    co0 = pltpu.async_copy(a0, out_hbm.at[pl.ds(base, CHUNK)], so)
    ga1.wait()
    gb1.wait()
    _add_rows(a1, b1)
    co0.wait()
    co1 = pltpu.async_copy(a1, out_hbm.at[pl.ds(base + CHUNK, CHUNK)], so)
    co1.wait()


def kernel(input_ids, position_ids, wte, wpe):
    idx = jnp.concatenate(
        [input_ids.reshape(-1), position_ids.reshape(-1)]
    ).astype(jnp.int32).reshape(2 * NW * 2, CHUNK)
    out = _embed(idx, wte, wpe)
    return out.reshape(input_ids.shape + (wte.shape[1],))


# speedup vs baseline: 1.0012x; 1.0012x over previous
import functools

import jax
import jax.numpy as jnp
from jax import lax
from jax.experimental import pallas as pl
from jax.experimental.pallas import tpu as pltpu
from jax.experimental.pallas import tpu_sc as plsc

D = 128
N_TOK = 8192
NC = 2
NS = 16
NW = NC * NS
PER_W = N_TOK // NW
SUB = 64
NSUB = PER_W // SUB
LANES = 16

_mesh = plsc.VectorSubcoreMesh(core_axis_name="c", subcore_axis_name="s")


def _add_rows(a, b, lo, hi):
    @plsc.parallel_loop(lo, hi, step=1, unroll=4)
    def body(i):
        for j in range(D // LANES):
            sl = pl.ds(j * LANES, LANES)
            plsc.addupdate(a.at[i, sl], b[i, sl])


@functools.partial(
    pl.kernel,
    out_type=jax.ShapeDtypeStruct((N_TOK, D), jnp.float32),
    mesh=_mesh,
    scratch_types=[
        pltpu.VMEM((2, 128), jnp.int32),
        pltpu.VMEM((2, 128), jnp.int32),
        pltpu.VMEM((PER_W, D), jnp.float32),
        pltpu.VMEM((PER_W, D), jnp.float32),
        pltpu.SemaphoreType.DMA,
        pltpu.SemaphoreType.DMA,
        pltpu.SemaphoreType.DMA,
        pltpu.SemaphoreType.DMA,
        pltpu.SemaphoreType.DMA,
        pltpu.SemaphoreType.DMA,
        pltpu.SemaphoreType.DMA,
        pltpu.SemaphoreType.DMA,
        pltpu.SemaphoreType.DMA,
        pltpu.SemaphoreType.DMA,
        pltpu.SemaphoreType.DMA,
    ],
)
def _embed(idx_hbm, wte_hbm, wpe_hbm, out_hbm,
           ti_v, pi_v, a, b,
           si0, si1, sa0, sa1, sa2, sa3, sb0, sb1, sb2, sb3, so):
    wid = lax.axis_index("s") * NC + lax.axis_index("c")
    r = wid * 2
    ci0 = pltpu.async_copy(idx_hbm.at[pl.ds(r, 2)], ti_v, si0)
    ci1 = pltpu.async_copy(idx_hbm.at[pl.ds(NW * 2 + r, 2)], pi_v, si1)
    ci0.wait()
    ci1.wait()
    sas = (sa0, sa1, sa2, sa3)
    sbs = (sb0, sb1, sb2, sb3)
    gas, gbs = [], []
    for q in range(NSUB):
        row, off = q // 2, (q % 2) * SUB
        dst = pl.ds(q * SUB, SUB)
        gas.append(pltpu.async_copy(
            wte_hbm.at[ti_v.at[row, pl.ds(off, SUB)]], a.at[dst], sas[q]))
        gbs.append(pltpu.async_copy(
            wpe_hbm.at[pi_v.at[row, pl.ds(off, SUB)]], b.at[dst], sbs[q]))
    base = wid * PER_W
    cos = []
    for q in range(NSUB):
        gas[q].wait()
        gbs[q].wait()
        _add_rows(a, b, q * SUB, (q + 1) * SUB)
        cos.append(pltpu.async_copy(
            a.at[pl.ds(q * SUB, SUB)],
            out_hbm.at[pl.ds(base + q * SUB, SUB)], so))
    for co in cos:
        co.wait()


def kernel(input_ids, position_ids, wte, wpe):
    idx = jnp.concatenate(
        [input_ids.reshape(-1), position_ids.reshape(-1)]
    ).astype(jnp.int32).reshape(2 * NW * 2, 128)
    out = _embed(idx, wte, wpe)
    return out.reshape(input_ids.shape + (wte.shape[1],))


# speedup vs baseline: 1.0052x; 1.0039x over previous
import functools

import jax
import jax.numpy as jnp
from jax import lax
from jax.experimental import pallas as pl
from jax.experimental.pallas import tpu as pltpu
from jax.experimental.pallas import tpu_sc as plsc

D = 128
BATCH = 4
SEQ = 2048
N_TOK = BATCH * SEQ
NC = 2
NS = 16
NW = NC * NS
PER_W = N_TOK // NW
W_PER_ROW = SEQ // PER_W
SUB = 64
NSUB = PER_W // SUB
LANES = 16

_mesh = plsc.VectorSubcoreMesh(core_axis_name="c", subcore_axis_name="s")


def _add_rows(a, b, lo, hi):
    @plsc.parallel_loop(lo, hi, step=1, unroll=4)
    def body(i):
        for j in range(D // LANES):
            sl = pl.ds(j * LANES, LANES)
            plsc.addupdate(a.at[i, sl], b[i, sl])


@functools.partial(
    pl.kernel,
    out_type=jax.ShapeDtypeStruct((N_TOK, D), jnp.float32),
    mesh=_mesh,
    scratch_types=[
        pltpu.VMEM((PER_W,), jnp.int32),
        pltpu.VMEM((PER_W,), jnp.int32),
        pltpu.VMEM((PER_W, D), jnp.float32),
        pltpu.VMEM((PER_W, D), jnp.float32),
        pltpu.SemaphoreType.DMA,
        pltpu.SemaphoreType.DMA,
        pltpu.SemaphoreType.DMA,
        pltpu.SemaphoreType.DMA,
        pltpu.SemaphoreType.DMA,
        pltpu.SemaphoreType.DMA,
        pltpu.SemaphoreType.DMA,
        pltpu.SemaphoreType.DMA,
        pltpu.SemaphoreType.DMA,
        pltpu.SemaphoreType.DMA,
        pltpu.SemaphoreType.DMA,
    ],
)
def _embed(ids_hbm, pos_hbm, wte_hbm, wpe_hbm, out_hbm,
           ti_v, pi_v, a, b,
           si0, si1, sa0, sa1, sa2, sa3, sb0, sb1, sb2, sb3, so):
    wid = lax.axis_index("s") * NC + lax.axis_index("c")
    brow = wid // W_PER_ROW
    s0 = (wid % W_PER_ROW) * PER_W
    ci0 = pltpu.async_copy(ids_hbm.at[brow, pl.ds(s0, PER_W)], ti_v, si0)
    ci1 = pltpu.async_copy(pos_hbm.at[brow, pl.ds(s0, PER_W)], pi_v, si1)
    ci0.wait()
    ci1.wait()
    sas = (sa0, sa1, sa2, sa3)
    sbs = (sb0, sb1, sb2, sb3)
    gas, gbs = [], []
    for q in range(NSUB):
        dst = pl.ds(q * SUB, SUB)
        gas.append(pltpu.async_copy(
            wte_hbm.at[ti_v.at[pl.ds(q * SUB, SUB)]], a.at[dst], sas[q]))
        gbs.append(pltpu.async_copy(
            wpe_hbm.at[pi_v.at[pl.ds(q * SUB, SUB)]], b.at[dst], sbs[q]))
    base = wid * PER_W
    cos = []
    for q in range(NSUB):
        gas[q].wait()
        gbs[q].wait()
        _add_rows(a, b, q * SUB, (q + 1) * SUB)
        cos.append(pltpu.async_copy(
            a.at[pl.ds(q * SUB, SUB)],
            out_hbm.at[pl.ds(base + q * SUB, SUB)], so))
    for co in cos:
        co.wait()


def kernel(input_ids, position_ids, wte, wpe):
    out = _embed(input_ids.astype(jnp.int32), position_ids.astype(jnp.int32),
                 wte, wpe)
    return out.reshape(input_ids.shape + (wte.shape[1],))


# speedup vs baseline: 1.0292x; 1.0239x over previous
import functools

import jax
import jax.numpy as jnp
from jax import lax
from jax.experimental import pallas as pl
from jax.experimental.pallas import tpu as pltpu
from jax.experimental.pallas import tpu_sc as plsc

D = 128
BATCH = 4
SEQ = 2048
N_TOK = BATCH * SEQ
NC = 2
NS = 16
NW = NC * NS
PER_W = N_TOK // NW
W_PER_ROW = SEQ // PER_W
SUB = 64
NSUB = PER_W // SUB

_mesh = plsc.VectorSubcoreMesh(core_axis_name="c", subcore_axis_name="s")


@functools.partial(
    pl.kernel,
    out_type=jax.ShapeDtypeStruct((N_TOK, D), jnp.float32),
    mesh=_mesh,
    scratch_types=[
        pltpu.VMEM((PER_W,), jnp.int32),
        pltpu.VMEM((PER_W,), jnp.int32),
        pltpu.VMEM((PER_W, D), jnp.float32),
        pltpu.SemaphoreType.DMA,
        pltpu.SemaphoreType.DMA,
        pltpu.SemaphoreType.DMA,
        pltpu.SemaphoreType.DMA,
        pltpu.SemaphoreType.DMA,
        pltpu.SemaphoreType.DMA,
        pltpu.SemaphoreType.DMA,
        pltpu.SemaphoreType.DMA,
        pltpu.SemaphoreType.DMA,
        pltpu.SemaphoreType.DMA,
        pltpu.SemaphoreType.DMA,
    ],
)
def _embed(ids_hbm, pos_hbm, wte_hbm, wpe_hbm, out_hbm,
           ti_v, pi_v, a,
           si0, si1, sa0, sa1, sa2, sa3, sb0, sb1, sb2, sb3, so):
    wid = lax.axis_index("s") * NC + lax.axis_index("c")
    brow = wid // W_PER_ROW
    s0 = (wid % W_PER_ROW) * PER_W
    ci0 = pltpu.async_copy(ids_hbm.at[brow, pl.ds(s0, PER_W)], ti_v, si0)
    ci1 = pltpu.async_copy(pos_hbm.at[brow, pl.ds(s0, PER_W)], pi_v, si1)
    ci0.wait()
    ci1.wait()
    sas = (sa0, sa1, sa2, sa3)
    sbs = (sb0, sb1, sb2, sb3)
    gas = []
    for q in range(NSUB):
        gas.append(pltpu.async_copy(
            wte_hbm.at[ti_v.at[pl.ds(q * SUB, SUB)]],
            a.at[pl.ds(q * SUB, SUB)], sas[q]))
    gbs = []
    for q in range(NSUB):
        gas[q].wait()
        gbs.append(pltpu.async_copy(
            wpe_hbm.at[pi_v.at[pl.ds(q * SUB, SUB)]],
            a.at[pl.ds(q * SUB, SUB)], sbs[q], add=True))
    base = wid * PER_W
    cos = []
    for q in range(NSUB):
        gbs[q].wait()
        cos.append(pltpu.async_copy(
            a.at[pl.ds(q * SUB, SUB)],
            out_hbm.at[pl.ds(base + q * SUB, SUB)], so))
    for co in cos:
        co.wait()


def kernel(input_ids, position_ids, wte, wpe):
    out = _embed(input_ids.astype(jnp.int32), position_ids.astype(jnp.int32),
                 wte, wpe)
    return out.reshape(input_ids.shape + (wte.shape[1],))
